# baseline (speedup 1.0000x reference)
import numpy as np
import jax
import jax.numpy as jnp
from jax import lax
from jax.experimental import pallas as pl
from jax.experimental.pallas import tpu as pltpu

R = 32
C = 800000
BLOCK_C = 16000
GRID = C // BLOCK_C

_TINY = np.float32(np.finfo(np.float32).tiny)
_SCALE = np.float32(np.float32(1.0) - _TINY)

_KS0 = np.uint32(0)
_KS1 = np.uint32(42)
_KS2 = np.uint32(0x1BD11BDA) ^ _KS0 ^ _KS1
_ROT = ((13, 15, 26, 6), (17, 29, 16, 24))


def _threefry_bits(idx_u32):
    ks = (_KS0, _KS1, _KS2)
    x0 = jnp.full_like(idx_u32, _KS0)
    x1 = idx_u32 + _KS1
    for i in range(5):
        for r in _ROT[i % 2]:
            x0 = x0 + x1
            x1 = (x1 << np.uint32(r)) | (x1 >> np.uint32(32 - r))
            x1 = x0 ^ x1
        x0 = x0 + ks[(i + 1) % 3]
        x1 = x1 + ks[(i + 2) % 3] + np.uint32(i + 1)
    return x0 ^ x1


def _gumbel_from_bits(bits):
    fb = (bits >> np.uint32(9)) | np.uint32(0x3F800000)
    floats = lax.bitcast_convert_type(fb, jnp.float32) - np.float32(1.0)
    u = jnp.maximum(_TINY, floats * _SCALE + _TINY)
    return -jnp.log(-jnp.log(u))


def _full_pass_kernel(p_ref, val_out, idx_out, best_val, best_idx):
    c = pl.program_id(0)
    base = c * BLOCK_C

    row = lax.broadcasted_iota(jnp.int32, (R, BLOCK_C), 0)
    col = lax.broadcasted_iota(jnp.int32, (R, BLOCK_C), 1)
    flat = (row * C + col + base).astype(jnp.uint32)

    g = _gumbel_from_bits(_threefry_bits(flat))
    val = jnp.log(p_ref[...] + np.float32(1e-30)) + g

    m = jnp.max(val, axis=1, keepdims=True)
    col_glob = col + base
    big = jnp.int32(2**31 - 1)
    idx = jnp.min(jnp.where(val == m, col_glob, big), axis=1, keepdims=True)

    @pl.when(c == 0)
    def _init():
        best_val[...] = m
        best_idx[...] = idx

    @pl.when(c != 0)
    def _update():
        upd = m > best_val[...]
        best_val[...] = jnp.where(upd, m, best_val[...])
        best_idx[...] = jnp.where(upd, idx, best_idx[...])

    @pl.when(c == GRID - 1)
    def _finish():
        val_out[...] = best_val[...]
        idx_out[...] = best_idx[...]


@jax.jit
def _sample_full(p_flat_2d):
    _, idx = pl.pallas_call(
        _full_pass_kernel,
        grid=(GRID,),
        in_specs=[pl.BlockSpec((R, BLOCK_C), lambda c: (0, c))],
        out_specs=[
            pl.BlockSpec((R, 1), lambda c: (0, 0)),
            pl.BlockSpec((R, 1), lambda c: (0, 0)),
        ],
        out_shape=[
            jax.ShapeDtypeStruct((R, 1), jnp.float32),
            jax.ShapeDtypeStruct((R, 1), jnp.int32),
        ],
        scratch_shapes=[
            pltpu.VMEM((R, 1), jnp.float32),
            pltpu.VMEM((R, 1), jnp.int32),
        ],
    )(p_flat_2d)
    return idx[:, 0]


def kernel(action_generation_output, action_probability_output):
    del action_generation_output
    batch, seq, _ = action_probability_output.shape
    p2d = action_probability_output.reshape(R, C)
    idx = _sample_full(p2d)
    return idx.reshape(batch, seq // batch).astype(jnp.int32)
